# Initial kernel scaffold; baseline (speedup 1.0000x reference)
import functools

import jax
import jax.numpy as jnp
from jax import lax
from jax.experimental import pallas as pl
from jax.experimental.pallas import tpu as pltpu
from jax.experimental.pallas import tpu_sc as plsc

N = 10000
E = 160000
B = 4
K = 4
NB = N * B

CH = 128
EP = 163840
NCHUNKS = EP // CH
NWORKERS = 32
CPT = NCHUNKS // NWORKERS
NTILES = 16
RPT = N // NTILES

BLK = 800
BLKN = 1000
EPS = 1e-5


@functools.cache
def _make_spmm(F):
  mesh = plsc.VectorSubcoreMesh(core_axis_name="c", subcore_axis_name="s")

  def body(x_hbm, src_hbm, dst_hbm, w_hbm, z_hbm, out_hbm,
           src_v, dst_v, w_v, rows_v, acc, sem):
    cid = lax.axis_index("c")
    sid = lax.axis_index("s")
    wid = sid * 2 + cid
    r0 = sid * RPT
    pltpu.sync_copy(z_hbm.at[pl.ds(r0, RPT)], acc.at[pl.ds(r0, RPT)])
    c0 = wid * CPT
    pltpu.sync_copy(src_hbm.at[pl.ds(c0, CPT)], src_v)
    pltpu.sync_copy(dst_hbm.at[pl.ds(c0, CPT)], dst_v)
    pltpu.sync_copy(w_hbm.at[pl.ds(c0, CPT)], w_v)
    plsc.subcore_barrier()

    def chunk_body(kk, carry):
      pltpu.async_copy(x_hbm.at[src_v.at[kk]], rows_v, sem).wait()

      def row_body(r, c2):
        wbc = plsc.load_gather(
            w_v, [jnp.full((16,), kk, jnp.int32), jnp.full((16,), r, jnp.int32)])
        for j in range(F // 16):
          sl = pl.ds(j * 16, 16)
          rows_v[r, sl] = rows_v[r, sl] * wbc
        return c2

      lax.fori_loop(0, CH, row_body, 0)
      pltpu.sync_copy(rows_v, acc.at[dst_v.at[kk]], add=True)
      return carry

    lax.fori_loop(0, CPT, chunk_body, 0)
    plsc.subcore_barrier()
    pltpu.sync_copy(acc.at[pl.ds(r0, RPT)], out_hbm.at[cid, pl.ds(r0, RPT)])

  return pl.kernel(
      body,
      out_type=jax.ShapeDtypeStruct((2, N, F), jnp.float32),
      mesh=mesh,
      scratch_types=[
          pltpu.VMEM((CPT, CH), jnp.int32),
          pltpu.VMEM((CPT, CH), jnp.int32),
          pltpu.VMEM((CPT, CH), jnp.float32),
          pltpu.VMEM((CH, F), jnp.float32),
          pltpu.VMEM_SHARED((N, F), jnp.float32),
          pltpu.SemaphoreType.DMA,
      ],
  )


@functools.cache
def _make_cheb_first(F):
  def body(p_ref, o_ref):
    o_ref[...] = p_ref[0] + p_ref[1]

  return pl.pallas_call(
      body,
      grid=(N // BLKN,),
      in_specs=[pl.BlockSpec((2, BLKN, F), lambda i: (0, i, 0))],
      out_specs=pl.BlockSpec((BLKN, F), lambda i: (i, 0)),
      out_shape=jax.ShapeDtypeStruct((N, F), jnp.float32),
  )


@functools.cache
def _make_cheb_next(F):
  def body(p_ref, t_ref, o_ref):
    o_ref[...] = 2.0 * (p_ref[0] + p_ref[1]) - t_ref[...]

  return pl.pallas_call(
      body,
      grid=(N // BLKN,),
      in_specs=[
          pl.BlockSpec((2, BLKN, F), lambda i: (0, i, 0)),
          pl.BlockSpec((BLKN, F), lambda i: (i, 0)),
      ],
      out_specs=pl.BlockSpec((BLKN, F), lambda i: (i, 0)),
      out_shape=jax.ShapeDtypeStruct((N, F), jnp.float32),
  )


@functools.cache
def _make_conv_out(rows, C, F, shortcut, cs, relu, stats):
  grid = (rows // BLK,)

  def body(*refs):
    t0, t1, t2, p3, w, bv = refs[:6]
    i = 6
    if shortcut == "proj":
      s, ws, bsv = refs[i:i + 3]
      i += 3
    elif shortcut == "id":
      s = refs[i]
      i += 1
    y = refs[i]
    i += 1
    if stats:
      ssum, ssq = refs[i:i + 2]
    gi = pl.program_id(0)
    t3 = 2.0 * (p3[0] + p3[1]) - t1[...]
    acc = (jnp.dot(t0[...], w[0], preferred_element_type=jnp.float32)
           + jnp.dot(t1[...], w[1], preferred_element_type=jnp.float32)
           + jnp.dot(t2[...], w[2], preferred_element_type=jnp.float32)
           + jnp.dot(t3, w[3], preferred_element_type=jnp.float32))
    acc = acc + bv[...]
    if shortcut == "proj":
      acc = acc + jnp.dot(s[...], ws[...], preferred_element_type=jnp.float32) + bsv[...]
    elif shortcut == "id":
      acc = acc + s[...]
    if relu:
      acc = jnp.maximum(acc, 0.0)
    y[...] = acc
    if stats:
      ps = jnp.sum(acc, axis=0, keepdims=True)
      pq = jnp.sum(acc * acc, axis=0, keepdims=True)

      @pl.when(gi == 0)
      def _():
        ssum[...] = ps
        ssq[...] = pq

      @pl.when(gi != 0)
      def _():
        ssum[...] = ssum[...] + ps
        ssq[...] = ssq[...] + pq

  in_specs = [
      pl.BlockSpec((BLK, C), lambda i: (i, 0)),
      pl.BlockSpec((BLK, C), lambda i: (i, 0)),
      pl.BlockSpec((BLK, C), lambda i: (i, 0)),
      pl.BlockSpec((2, BLK, C), lambda i: (0, i, 0)),
      pl.BlockSpec((K, C, F), lambda i: (0, 0, 0)),
      pl.BlockSpec((1, F), lambda i: (0, 0)),
  ]
  if shortcut == "proj":
    in_specs += [
        pl.BlockSpec((BLK, cs), lambda i: (i, 0)),
        pl.BlockSpec((cs, F), lambda i: (0, 0)),
        pl.BlockSpec((1, F), lambda i: (0, 0)),
    ]
  elif shortcut == "id":
    in_specs += [pl.BlockSpec((BLK, F), lambda i: (i, 0))]
  out_specs = [pl.BlockSpec((BLK, F), lambda i: (i, 0))]
  out_shape = [jax.ShapeDtypeStruct((rows, F), jnp.float32)]
  if stats:
    out_specs += [pl.BlockSpec((1, F), lambda i: (0, 0))] * 2
    out_shape += [jax.ShapeDtypeStruct((1, F), jnp.float32)] * 2

  return pl.pallas_call(
      body,
      grid=grid,
      in_specs=in_specs,
      out_specs=out_specs,
      out_shape=out_shape,
  )


@functools.cache
def _make_bn_relu(C):
  def body(x_ref, s_ref, q_ref, g_ref, b_ref, o_ref):
    m = s_ref[...] / float(NB)
    v = q_ref[...] / float(NB) - m * m
    inv = lax.rsqrt(v + EPS)
    o_ref[...] = jnp.maximum((x_ref[...] - m) * inv * g_ref[...] + b_ref[...], 0.0)

  return pl.pallas_call(
      body,
      grid=(NB // BLK,),
      in_specs=[
          pl.BlockSpec((BLK, C), lambda i: (i, 0)),
          pl.BlockSpec((1, C), lambda i: (0, 0)),
          pl.BlockSpec((1, C), lambda i: (0, 0)),
          pl.BlockSpec((1, C), lambda i: (0, 0)),
          pl.BlockSpec((1, C), lambda i: (0, 0)),
      ],
      out_specs=pl.BlockSpec((BLK, C), lambda i: (i, 0)),
      out_shape=jax.ShapeDtypeStruct((NB, C), jnp.float32),
  )


def _make_head():
  F = 64
  NC = 10
  grid_n = N // BLKN

  def body(y0, y1, y2, y3, fw, fb, o_ref, mx):
    gi = pl.program_id(0)
    cur = jnp.concatenate(
        [jnp.max(y[...], axis=0, keepdims=True) for y in (y0, y1, y2, y3)], axis=0)

    @pl.when(gi == 0)
    def _():
      mx[...] = cur

    @pl.when(gi != 0)
    def _():
      mx[...] = jnp.maximum(mx[...], cur)

    @pl.when(gi == grid_n - 1)
    def _():
      z = jnp.dot(mx[...], fw[...], preferred_element_type=jnp.float32) + fb[...]
      zm = jnp.max(z, axis=1, keepdims=True)
      e = jnp.exp(z - zm)
      o_ref[...] = (z - zm) - jnp.log(jnp.sum(e, axis=1, keepdims=True))

  return pl.pallas_call(
      body,
      grid=(grid_n,),
      in_specs=[pl.BlockSpec((BLKN, F), lambda i: (i, 0))] * 4 + [
          pl.BlockSpec((F, NC), lambda i: (0, 0)),
          pl.BlockSpec((1, NC), lambda i: (0, 0)),
      ],
      out_specs=pl.BlockSpec((B, NC), lambda i: (0, 0)),
      out_shape=jax.ShapeDtypeStruct((B, NC), jnp.float32),
      scratch_shapes=[pltpu.VMEM((B, F), jnp.float32)],
  )


def kernel(x, params, edge_src, edge_dst, edge_w):
  p = params
  src = edge_src.astype(jnp.int32)
  dst = edge_dst.astype(jnp.int32)
  w = edge_w.astype(jnp.float32)
  padn = EP - E
  pidx = jnp.arange(padn, dtype=jnp.int32) % N
  srcC = jnp.concatenate([src, pidx]).reshape(NCHUNKS, CH)
  dstC = jnp.concatenate([dst, pidx]).reshape(NCHUNKS, CH)
  wC = jnp.concatenate([w, jnp.zeros((padn,), jnp.float32)]).reshape(NCHUNKS, CH)
  zeros = {f: jnp.zeros((N, f), jnp.float32) for f in (16, 64, 128)}

  def spmm(xt):
    return _make_spmm(xt.shape[1])(xt, srcC, dstC, wC, zeros[xt.shape[1]])

  def cheb_T(xt):
    P1 = spmm(xt)
    T1 = _make_cheb_first(xt.shape[1])(P1)
    P2 = spmm(T1)
    T2 = _make_cheb_next(xt.shape[1])(P2, xt)
    P3 = spmm(T2)
    return xt, T1, T2, P3

  def conv(xt, rows, C, W, bias, shortcut="none", S=None, Ws=None, bs=None,
           relu=False, stats=True):
    T0, T1, T2, P3 = cheb_T(xt)
    F = W.shape[2]
    args = [T0.reshape(rows, C), T1.reshape(rows, C), T2.reshape(rows, C),
            P3.reshape(2, rows, C), W, bias.reshape(1, F)]
    if shortcut == "proj":
      args += [S, Ws, bs.reshape(1, F)]
    elif shortcut == "id":
      args += [S]
    return _make_conv_out(rows, C, F, shortcut, 0 if S is None else S.shape[1],
                          relu, stats)(*args)

  def bn_relu(h, ss, sq, g, b):
    C = h.shape[1]
    return _make_bn_relu(C)(h, ss, sq, g.reshape(1, C), b.reshape(1, C))

  xt16 = jnp.pad(jnp.transpose(x, (2, 0, 1)), ((0, 0), (0, 0), (0, 1))).reshape(N, 16)
  W0p = jnp.pad(p['conv0_W'], ((0, 0), (0, 1), (0, 0)))

  out0, s0, q0 = conv(xt16, NB, 4, W0p, p['conv0_b'], relu=True)

  a = bn_relu(out0, s0, q0, p['b1_bn1_g'], p['b1_bn1_b'])
  h1, hs, hq = conv(a.reshape(N, 64), NB, 16, p['b1_W1'], p['b1_b1'])
  a2 = bn_relu(h1, hs, hq, p['b1_bn2_g'], p['b1_bn2_b'])
  x1, s1, q1 = conv(a2.reshape(N, 64), NB, 16, p['b1_W2'], p['b1_b2'],
                    shortcut="id", S=out0)

  a = bn_relu(x1, s1, q1, p['b2_bn1_g'], p['b2_bn1_b'])
  h1, hs, hq = conv(a.reshape(N, 64), NB, 16, p['b2_W1'], p['b2_b1'])
  a2 = bn_relu(h1, hs, hq, p['b2_bn2_g'], p['b2_bn2_b'])
  x2, s2, q2 = conv(a2.reshape(N, 128), NB, 32, p['b2_W2'], p['b2_b2'],
                    shortcut="proj", S=a, Ws=p['b2_Ws'], bs=p['b2_bs'])

  a = bn_relu(x2, s2, q2, p['b3_bn1_g'], p['b3_bn1_b'])
  h1, hs, hq = conv(a.reshape(N, 128), NB, 32, p['b3_W1'], p['b3_b1'])
  a2 = bn_relu(h1, hs, hq, p['b3_bn2_g'], p['b3_bn2_b'])

  a2r = a2.reshape(N, B, 64)
  ar = a.reshape(N, B, 32)
  ybs = []
  for h in range(2):
    xt_h = a2r[:, 2 * h:2 * h + 2, :].reshape(N, 128)
    S_h = ar[:, 2 * h:2 * h + 2, :].reshape(2 * N, 32)
    y_h = conv(xt_h, 2 * N, 64, p['b3_W2'], p['b3_b2'],
               shortcut="proj", S=S_h, Ws=p['b3_Ws'], bs=p['b3_bs'], stats=False)
    yr = y_h.reshape(N, 2, 64)
    ybs += [yr[:, 0, :], yr[:, 1, :]]

  return _make_head()(ybs[0], ybs[1], ybs[2], ybs[3],
                      p['fc_W'], p['fc_b'].reshape(1, 10))


# speedup vs baseline: 3.9554x; 3.9554x over previous
import functools

import jax
import jax.numpy as jnp
from jax import lax
from jax.experimental import pallas as pl
from jax.experimental.pallas import tpu as pltpu
from jax.experimental.pallas import tpu_sc as plsc

N = 10000
E = 160000
B = 4
K = 4
NB = N * B

CH = 128
EP = 163840
NCHUNKS = EP // CH
NWORKERS = 32
CPT = NCHUNKS // NWORKERS
NTILES = 16
_ROWSPLIT = tuple((t * 632, 632 if t < 15 else N - 15 * 632) for t in range(NTILES))

BLK = 800
BLKN = 1000
EPS = 1e-5


@functools.cache
def _make_spmm(F):
  mesh = plsc.VectorSubcoreMesh(core_axis_name="c", subcore_axis_name="s")

  def body(x_hbm, src_hbm, dst_hbm, w_hbm, z_hbm, out_hbm,
           src_v, dst_v, w_v, rows_v, acc, sem):
    cid = lax.axis_index("c")
    sid = lax.axis_index("s")
    wid = sid * 2 + cid
    for t, (t0, tn) in enumerate(_ROWSPLIT):
      @pl.when(sid == t)
      def _(t0=t0, tn=tn):
        pltpu.sync_copy(z_hbm.at[pl.ds(t0, tn)], acc.at[pl.ds(t0, tn)])
    c0 = wid * CPT
    pltpu.sync_copy(src_hbm.at[pl.ds(c0, CPT)], src_v)
    pltpu.sync_copy(dst_hbm.at[pl.ds(c0, CPT)], dst_v)
    pltpu.sync_copy(w_hbm.at[pl.ds(c0, CPT)], w_v)
    plsc.subcore_barrier()

    def chunk_body(kk, carry):
      pltpu.async_copy(x_hbm.at[src_v.at[kk]], rows_v, sem).wait()

      def grp_body(g, c2):
        wg = w_v[kk, pl.ds(g * 16, 16)]
        row0 = g * 16
        for r in range(16):
          wbc = jnp.broadcast_to(wg[r], (16,))
          for j in range(F // 16):
            sl = pl.ds(j * 16, 16)
            rows_v[row0 + r, sl] = rows_v[row0 + r, sl] * wbc
        return c2

      lax.fori_loop(0, CH // 16, grp_body, 0)
      pltpu.sync_copy(rows_v, acc.at[dst_v.at[kk]], add=True)
      return carry

    lax.fori_loop(0, CPT, chunk_body, 0)
    plsc.subcore_barrier()
    for t, (t0, tn) in enumerate(_ROWSPLIT):
      @pl.when(sid == t)
      def _(t0=t0, tn=tn):
        pltpu.sync_copy(acc.at[pl.ds(t0, tn)], out_hbm.at[cid, pl.ds(t0, tn)])

  return pl.kernel(
      body,
      out_type=jax.ShapeDtypeStruct((2, N, F), jnp.float32),
      mesh=mesh,
      compiler_params=pltpu.CompilerParams(use_tc_tiling_on_sc=False),
      scratch_types=[
          pltpu.VMEM((CPT, CH), jnp.int32),
          pltpu.VMEM((CPT, CH), jnp.int32),
          pltpu.VMEM((CPT, CH), jnp.float32),
          pltpu.VMEM((CH, F), jnp.float32),
          pltpu.VMEM_SHARED((N, F), jnp.float32),
          pltpu.SemaphoreType.DMA,
      ],
  )


@functools.cache
def _make_cheb_first(F):
  def body(p_ref, o_ref):
    o_ref[...] = p_ref[0] + p_ref[1]

  return pl.pallas_call(
      body,
      grid=(N // BLKN,),
      in_specs=[pl.BlockSpec((2, BLKN, F), lambda i: (0, i, 0))],
      out_specs=pl.BlockSpec((BLKN, F), lambda i: (i, 0)),
      out_shape=jax.ShapeDtypeStruct((N, F), jnp.float32),
  )


@functools.cache
def _make_cheb_next(F):
  def body(p_ref, t_ref, o_ref):
    o_ref[...] = 2.0 * (p_ref[0] + p_ref[1]) - t_ref[...]

  return pl.pallas_call(
      body,
      grid=(N // BLKN,),
      in_specs=[
          pl.BlockSpec((2, BLKN, F), lambda i: (0, i, 0)),
          pl.BlockSpec((BLKN, F), lambda i: (i, 0)),
      ],
      out_specs=pl.BlockSpec((BLKN, F), lambda i: (i, 0)),
      out_shape=jax.ShapeDtypeStruct((N, F), jnp.float32),
  )


@functools.cache
def _make_conv_out(rows, C, F, shortcut, cs, relu, stats):
  grid = (rows // BLK,)

  def body(*refs):
    t0, t1, t2, p3, w, bv = refs[:6]
    i = 6
    if shortcut == "proj":
      s, ws, bsv = refs[i:i + 3]
      i += 3
    elif shortcut == "id":
      s = refs[i]
      i += 1
    y = refs[i]
    i += 1
    if stats:
      ssum, ssq = refs[i:i + 2]
    gi = pl.program_id(0)
    t3 = 2.0 * (p3[0] + p3[1]) - t1[...]
    acc = (jnp.dot(t0[...], w[0], preferred_element_type=jnp.float32)
           + jnp.dot(t1[...], w[1], preferred_element_type=jnp.float32)
           + jnp.dot(t2[...], w[2], preferred_element_type=jnp.float32)
           + jnp.dot(t3, w[3], preferred_element_type=jnp.float32))
    acc = acc + bv[...]
    if shortcut == "proj":
      acc = acc + jnp.dot(s[...], ws[...], preferred_element_type=jnp.float32) + bsv[...]
    elif shortcut == "id":
      acc = acc + s[...]
    if relu:
      acc = jnp.maximum(acc, 0.0)
    y[...] = acc
    if stats:
      ps = jnp.sum(acc, axis=0, keepdims=True)
      pq = jnp.sum(acc * acc, axis=0, keepdims=True)

      @pl.when(gi == 0)
      def _():
        ssum[...] = ps
        ssq[...] = pq

      @pl.when(gi != 0)
      def _():
        ssum[...] = ssum[...] + ps
        ssq[...] = ssq[...] + pq

  in_specs = [
      pl.BlockSpec((BLK, C), lambda i: (i, 0)),
      pl.BlockSpec((BLK, C), lambda i: (i, 0)),
      pl.BlockSpec((BLK, C), lambda i: (i, 0)),
      pl.BlockSpec((2, BLK, C), lambda i: (0, i, 0)),
      pl.BlockSpec((K, C, F), lambda i: (0, 0, 0)),
      pl.BlockSpec((1, F), lambda i: (0, 0)),
  ]
  if shortcut == "proj":
    in_specs += [
        pl.BlockSpec((BLK, cs), lambda i: (i, 0)),
        pl.BlockSpec((cs, F), lambda i: (0, 0)),
        pl.BlockSpec((1, F), lambda i: (0, 0)),
    ]
  elif shortcut == "id":
    in_specs += [pl.BlockSpec((BLK, F), lambda i: (i, 0))]
  out_specs = [pl.BlockSpec((BLK, F), lambda i: (i, 0))]
  out_shape = [jax.ShapeDtypeStruct((rows, F), jnp.float32)]
  if stats:
    out_specs += [pl.BlockSpec((1, F), lambda i: (0, 0))] * 2
    out_shape += [jax.ShapeDtypeStruct((1, F), jnp.float32)] * 2

  return pl.pallas_call(
      body,
      grid=grid,
      in_specs=in_specs,
      out_specs=out_specs,
      out_shape=out_shape,
  )


@functools.cache
def _make_bn_relu(C):
  def body(x_ref, s_ref, q_ref, g_ref, b_ref, o_ref):
    m = s_ref[...] / float(NB)
    v = q_ref[...] / float(NB) - m * m
    inv = lax.rsqrt(v + EPS)
    o_ref[...] = jnp.maximum((x_ref[...] - m) * inv * g_ref[...] + b_ref[...], 0.0)

  return pl.pallas_call(
      body,
      grid=(NB // BLK,),
      in_specs=[
          pl.BlockSpec((BLK, C), lambda i: (i, 0)),
          pl.BlockSpec((1, C), lambda i: (0, 0)),
          pl.BlockSpec((1, C), lambda i: (0, 0)),
          pl.BlockSpec((1, C), lambda i: (0, 0)),
          pl.BlockSpec((1, C), lambda i: (0, 0)),
      ],
      out_specs=pl.BlockSpec((BLK, C), lambda i: (i, 0)),
      out_shape=jax.ShapeDtypeStruct((NB, C), jnp.float32),
  )


def _make_head():
  F = 64
  NC = 10
  grid_n = N // BLKN

  def body(y0, y1, y2, y3, fw, fb, o_ref, mx):
    gi = pl.program_id(0)
    cur = jnp.concatenate(
        [jnp.max(y[...], axis=0, keepdims=True) for y in (y0, y1, y2, y3)], axis=0)

    @pl.when(gi == 0)
    def _():
      mx[...] = cur

    @pl.when(gi != 0)
    def _():
      mx[...] = jnp.maximum(mx[...], cur)

    @pl.when(gi == grid_n - 1)
    def _():
      z = jnp.dot(mx[...], fw[...], preferred_element_type=jnp.float32) + fb[...]
      zm = jnp.max(z, axis=1, keepdims=True)
      e = jnp.exp(z - zm)
      o_ref[...] = (z - zm) - jnp.log(jnp.sum(e, axis=1, keepdims=True))

  return pl.pallas_call(
      body,
      grid=(grid_n,),
      in_specs=[pl.BlockSpec((BLKN, F), lambda i: (i, 0))] * 4 + [
          pl.BlockSpec((F, NC), lambda i: (0, 0)),
          pl.BlockSpec((1, NC), lambda i: (0, 0)),
      ],
      out_specs=pl.BlockSpec((B, NC), lambda i: (0, 0)),
      out_shape=jax.ShapeDtypeStruct((B, NC), jnp.float32),
      scratch_shapes=[pltpu.VMEM((B, F), jnp.float32)],
  )


def kernel(x, params, edge_src, edge_dst, edge_w):
  p = params
  src = edge_src.astype(jnp.int32)
  dst = edge_dst.astype(jnp.int32)
  w = edge_w.astype(jnp.float32)
  padn = EP - E
  pidx = jnp.arange(padn, dtype=jnp.int32) % N
  srcC = jnp.concatenate([src, pidx]).reshape(NCHUNKS, CH)
  dstC = jnp.concatenate([dst, pidx]).reshape(NCHUNKS, CH)
  wC = jnp.concatenate([w, jnp.zeros((padn,), jnp.float32)]).reshape(NCHUNKS, CH)
  zeros = {f: jnp.zeros((N, f), jnp.float32) for f in (16, 64, 128)}

  def spmm(xt):
    return _make_spmm(xt.shape[1])(xt, srcC, dstC, wC, zeros[xt.shape[1]])

  def cheb_T(xt):
    P1 = spmm(xt)
    T1 = _make_cheb_first(xt.shape[1])(P1)
    P2 = spmm(T1)
    T2 = _make_cheb_next(xt.shape[1])(P2, xt)
    P3 = spmm(T2)
    return xt, T1, T2, P3

  def conv(xt, rows, C, W, bias, shortcut="none", S=None, Ws=None, bs=None,
           relu=False, stats=True):
    T0, T1, T2, P3 = cheb_T(xt)
    F = W.shape[2]
    args = [T0.reshape(rows, C), T1.reshape(rows, C), T2.reshape(rows, C),
            P3.reshape(2, rows, C), W, bias.reshape(1, F)]
    if shortcut == "proj":
      args += [S, Ws, bs.reshape(1, F)]
    elif shortcut == "id":
      args += [S]
    res = _make_conv_out(rows, C, F, shortcut, 0 if S is None else S.shape[1],
                         relu, stats)(*args)
    return res if stats else res[0]

  def bn_relu(h, ss, sq, g, b):
    C = h.shape[1]
    return _make_bn_relu(C)(h, ss, sq, g.reshape(1, C), b.reshape(1, C))

  xt16 = jnp.pad(jnp.transpose(x, (2, 0, 1)), ((0, 0), (0, 0), (0, 1))).reshape(N, 16)
  W0p = jnp.pad(p['conv0_W'], ((0, 0), (0, 1), (0, 0)))

  out0, s0, q0 = conv(xt16, NB, 4, W0p, p['conv0_b'], relu=True)

  a = bn_relu(out0, s0, q0, p['b1_bn1_g'], p['b1_bn1_b'])
  h1, hs, hq = conv(a.reshape(N, 64), NB, 16, p['b1_W1'], p['b1_b1'])
  a2 = bn_relu(h1, hs, hq, p['b1_bn2_g'], p['b1_bn2_b'])
  x1, s1, q1 = conv(a2.reshape(N, 64), NB, 16, p['b1_W2'], p['b1_b2'],
                    shortcut="id", S=out0)

  a = bn_relu(x1, s1, q1, p['b2_bn1_g'], p['b2_bn1_b'])
  h1, hs, hq = conv(a.reshape(N, 64), NB, 16, p['b2_W1'], p['b2_b1'])
  a2 = bn_relu(h1, hs, hq, p['b2_bn2_g'], p['b2_bn2_b'])
  x2, s2, q2 = conv(a2.reshape(N, 128), NB, 32, p['b2_W2'], p['b2_b2'],
                    shortcut="proj", S=a, Ws=p['b2_Ws'], bs=p['b2_bs'])

  a = bn_relu(x2, s2, q2, p['b3_bn1_g'], p['b3_bn1_b'])
  h1, hs, hq = conv(a.reshape(N, 128), NB, 32, p['b3_W1'], p['b3_b1'])
  a2 = bn_relu(h1, hs, hq, p['b3_bn2_g'], p['b3_bn2_b'])

  a2r = a2.reshape(N, B, 64)
  ar = a.reshape(N, B, 32)
  ybs = []
  for h in range(2):
    xt_h = a2r[:, 2 * h:2 * h + 2, :].reshape(N, 128)
    S_h = ar[:, 2 * h:2 * h + 2, :].reshape(2 * N, 32)
    y_h = conv(xt_h, 2 * N, 64, p['b3_W2'], p['b3_b2'],
               shortcut="proj", S=S_h, Ws=p['b3_Ws'], bs=p['b3_bs'], stats=False)
    yr = y_h.reshape(N, 2, 64)
    ybs += [yr[:, 0, :], yr[:, 1, :]]

  return _make_head()(ybs[0], ybs[1], ybs[2], ybs[3],
                      p['fc_W'], p['fc_b'].reshape(1, 10))


# speedup vs baseline: 5.0732x; 1.2826x over previous
import functools

import jax
import jax.numpy as jnp
from jax import lax
from jax.experimental import pallas as pl
from jax.experimental.pallas import tpu as pltpu
from jax.experimental.pallas import tpu_sc as plsc

N = 10000
E = 160000
B = 4
K = 4
NB = N * B

CH = 128
EP = 163840
NCHUNKS = EP // CH
NWORKERS = 32
CPT = NCHUNKS // NWORKERS
NTILES = 16
_ROWSPLIT = tuple((t * 632, 632 if t < 15 else N - 15 * 632) for t in range(NTILES))

BLK = 800
BLKN = 1000
EPS = 1e-5


@functools.cache
def _make_spmm(F):
  mesh = plsc.VectorSubcoreMesh(core_axis_name="c", subcore_axis_name="s")

  def body(x_hbm, src_hbm, dst_hbm, w_hbm, z_hbm, out_hbm,
           src_v, dst_v, w_v, rows0, rows1, acc, sem0, sem1):
    cid = lax.axis_index("c")
    sid = lax.axis_index("s")
    wid = sid * 2 + cid
    for t, (t0, tn) in enumerate(_ROWSPLIT):
      @pl.when(sid == t)
      def _(t0=t0, tn=tn):
        pltpu.sync_copy(z_hbm.at[pl.ds(t0, tn)], acc.at[pl.ds(t0, tn)])
    c0 = wid * CPT
    pltpu.sync_copy(src_hbm.at[pl.ds(c0, CPT)], src_v)
    pltpu.sync_copy(dst_hbm.at[pl.ds(c0, CPT)], dst_v)
    pltpu.sync_copy(w_hbm.at[pl.ds(c0, CPT)], w_v)
    plsc.subcore_barrier()

    def mult(rows, kk):
      def grp_body(g, c2):
        wg = w_v[kk, pl.ds(g * 16, 16)]
        row0 = g * 16
        for r in range(16):
          wbc = jnp.broadcast_to(wg[r], (16,))
          for j in range(F // 16):
            sl = pl.ds(j * 16, 16)
            rows[row0 + r, sl] = rows[row0 + r, sl] * wbc
        return c2

      lax.fori_loop(0, CH // 16, grp_body, 0)

    pltpu.async_copy(x_hbm.at[src_v.at[0]], rows0, sem0)

    def pair_body(m, carry):
      k0 = 2 * m
      pltpu.async_copy(x_hbm.at[src_v.at[k0 + 1]], rows1, sem1)
      pltpu.make_async_copy(x_hbm.at[src_v.at[k0]], rows0, sem0).wait()
      mult(rows0, k0)
      pltpu.sync_copy(rows0, acc.at[dst_v.at[k0]], add=True)
      knext = lax.rem(k0 + 2, CPT)
      pltpu.async_copy(x_hbm.at[src_v.at[knext]], rows0, sem0)
      pltpu.make_async_copy(x_hbm.at[src_v.at[k0 + 1]], rows1, sem1).wait()
      mult(rows1, k0 + 1)
      pltpu.sync_copy(rows1, acc.at[dst_v.at[k0 + 1]], add=True)
      return carry

    lax.fori_loop(0, CPT // 2, pair_body, 0)
    pltpu.make_async_copy(x_hbm.at[src_v.at[0]], rows0, sem0).wait()
    plsc.subcore_barrier()
    for t, (t0, tn) in enumerate(_ROWSPLIT):
      @pl.when(sid == t)
      def _(t0=t0, tn=tn):
        pltpu.sync_copy(acc.at[pl.ds(t0, tn)], out_hbm.at[cid, pl.ds(t0, tn)])

  return pl.kernel(
      body,
      out_type=jax.ShapeDtypeStruct((2, N, F), jnp.float32),
      mesh=mesh,
      compiler_params=pltpu.CompilerParams(use_tc_tiling_on_sc=False),
      scratch_types=[
          pltpu.VMEM((CPT, CH), jnp.int32),
          pltpu.VMEM((CPT, CH), jnp.int32),
          pltpu.VMEM((CPT, CH), jnp.float32),
          pltpu.VMEM((CH, F), jnp.float32),
          pltpu.VMEM((CH, F), jnp.float32),
          pltpu.VMEM_SHARED((N, F), jnp.float32),
          pltpu.SemaphoreType.DMA,
          pltpu.SemaphoreType.DMA,
      ],
  )


@functools.cache
def _make_cheb_first(F):
  def body(p_ref, o_ref):
    o_ref[...] = p_ref[0] + p_ref[1]

  return pl.pallas_call(
      body,
      grid=(N // BLKN,),
      in_specs=[pl.BlockSpec((2, BLKN, F), lambda i: (0, i, 0))],
      out_specs=pl.BlockSpec((BLKN, F), lambda i: (i, 0)),
      out_shape=jax.ShapeDtypeStruct((N, F), jnp.float32),
  )


@functools.cache
def _make_cheb_next(F):
  def body(p_ref, t_ref, o_ref):
    o_ref[...] = 2.0 * (p_ref[0] + p_ref[1]) - t_ref[...]

  return pl.pallas_call(
      body,
      grid=(N // BLKN,),
      in_specs=[
          pl.BlockSpec((2, BLKN, F), lambda i: (0, i, 0)),
          pl.BlockSpec((BLKN, F), lambda i: (i, 0)),
      ],
      out_specs=pl.BlockSpec((BLKN, F), lambda i: (i, 0)),
      out_shape=jax.ShapeDtypeStruct((N, F), jnp.float32),
  )


@functools.cache
def _make_conv_out(rows, C, F, shortcut, cs, relu, stats):
  grid = (rows // BLK,)

  def body(*refs):
    t0, t1, t2, p3, w, bv = refs[:6]
    i = 6
    if shortcut == "proj":
      s, ws, bsv = refs[i:i + 3]
      i += 3
    elif shortcut == "id":
      s = refs[i]
      i += 1
    y = refs[i]
    i += 1
    if stats:
      ssum, ssq = refs[i:i + 2]
    gi = pl.program_id(0)
    t3 = 2.0 * (p3[0] + p3[1]) - t1[...]
    acc = (jnp.dot(t0[...], w[0], preferred_element_type=jnp.float32)
           + jnp.dot(t1[...], w[1], preferred_element_type=jnp.float32)
           + jnp.dot(t2[...], w[2], preferred_element_type=jnp.float32)
           + jnp.dot(t3, w[3], preferred_element_type=jnp.float32))
    acc = acc + bv[...]
    if shortcut == "proj":
      acc = acc + jnp.dot(s[...], ws[...], preferred_element_type=jnp.float32) + bsv[...]
    elif shortcut == "id":
      acc = acc + s[...]
    if relu:
      acc = jnp.maximum(acc, 0.0)
    y[...] = acc
    if stats:
      ps = jnp.sum(acc, axis=0, keepdims=True)
      pq = jnp.sum(acc * acc, axis=0, keepdims=True)

      @pl.when(gi == 0)
      def _():
        ssum[...] = ps
        ssq[...] = pq

      @pl.when(gi != 0)
      def _():
        ssum[...] = ssum[...] + ps
        ssq[...] = ssq[...] + pq

  in_specs = [
      pl.BlockSpec((BLK, C), lambda i: (i, 0)),
      pl.BlockSpec((BLK, C), lambda i: (i, 0)),
      pl.BlockSpec((BLK, C), lambda i: (i, 0)),
      pl.BlockSpec((2, BLK, C), lambda i: (0, i, 0)),
      pl.BlockSpec((K, C, F), lambda i: (0, 0, 0)),
      pl.BlockSpec((1, F), lambda i: (0, 0)),
  ]
  if shortcut == "proj":
    in_specs += [
        pl.BlockSpec((BLK, cs), lambda i: (i, 0)),
        pl.BlockSpec((cs, F), lambda i: (0, 0)),
        pl.BlockSpec((1, F), lambda i: (0, 0)),
    ]
  elif shortcut == "id":
    in_specs += [pl.BlockSpec((BLK, F), lambda i: (i, 0))]
  out_specs = [pl.BlockSpec((BLK, F), lambda i: (i, 0))]
  out_shape = [jax.ShapeDtypeStruct((rows, F), jnp.float32)]
  if stats:
    out_specs += [pl.BlockSpec((1, F), lambda i: (0, 0))] * 2
    out_shape += [jax.ShapeDtypeStruct((1, F), jnp.float32)] * 2

  return pl.pallas_call(
      body,
      grid=grid,
      in_specs=in_specs,
      out_specs=out_specs,
      out_shape=out_shape,
  )


@functools.cache
def _make_bn_relu(C):
  def body(x_ref, s_ref, q_ref, g_ref, b_ref, o_ref):
    m = s_ref[...] / float(NB)
    v = q_ref[...] / float(NB) - m * m
    inv = lax.rsqrt(v + EPS)
    o_ref[...] = jnp.maximum((x_ref[...] - m) * inv * g_ref[...] + b_ref[...], 0.0)

  return pl.pallas_call(
      body,
      grid=(NB // BLK,),
      in_specs=[
          pl.BlockSpec((BLK, C), lambda i: (i, 0)),
          pl.BlockSpec((1, C), lambda i: (0, 0)),
          pl.BlockSpec((1, C), lambda i: (0, 0)),
          pl.BlockSpec((1, C), lambda i: (0, 0)),
          pl.BlockSpec((1, C), lambda i: (0, 0)),
      ],
      out_specs=pl.BlockSpec((BLK, C), lambda i: (i, 0)),
      out_shape=jax.ShapeDtypeStruct((NB, C), jnp.float32),
  )


def _make_head():
  F = 64
  NC = 10
  grid_n = N // BLKN

  def body(y0, y1, y2, y3, fw, fb, o_ref, mx):
    gi = pl.program_id(0)
    cur = jnp.concatenate(
        [jnp.max(y[...], axis=0, keepdims=True) for y in (y0, y1, y2, y3)], axis=0)

    @pl.when(gi == 0)
    def _():
      mx[...] = cur

    @pl.when(gi != 0)
    def _():
      mx[...] = jnp.maximum(mx[...], cur)

    @pl.when(gi == grid_n - 1)
    def _():
      z = jnp.dot(mx[...], fw[...], preferred_element_type=jnp.float32) + fb[...]
      zm = jnp.max(z, axis=1, keepdims=True)
      e = jnp.exp(z - zm)
      o_ref[...] = (z - zm) - jnp.log(jnp.sum(e, axis=1, keepdims=True))

  return pl.pallas_call(
      body,
      grid=(grid_n,),
      in_specs=[pl.BlockSpec((BLKN, F), lambda i: (i, 0))] * 4 + [
          pl.BlockSpec((F, NC), lambda i: (0, 0)),
          pl.BlockSpec((1, NC), lambda i: (0, 0)),
      ],
      out_specs=pl.BlockSpec((B, NC), lambda i: (0, 0)),
      out_shape=jax.ShapeDtypeStruct((B, NC), jnp.float32),
      scratch_shapes=[pltpu.VMEM((B, F), jnp.float32)],
  )


def kernel(x, params, edge_src, edge_dst, edge_w):
  p = params
  src = edge_src.astype(jnp.int32)
  dst = edge_dst.astype(jnp.int32)
  w = edge_w.astype(jnp.float32)
  padn = EP - E
  pidx = jnp.arange(padn, dtype=jnp.int32) % N
  srcC = jnp.concatenate([src, pidx]).reshape(NCHUNKS, CH)
  dstC = jnp.concatenate([dst, pidx]).reshape(NCHUNKS, CH)
  wC = jnp.concatenate([w, jnp.zeros((padn,), jnp.float32)]).reshape(NCHUNKS, CH)
  zeros = {f: jnp.zeros((N, f), jnp.float32) for f in (16, 64, 128)}

  def spmm(xt):
    return _make_spmm(xt.shape[1])(xt, srcC, dstC, wC, zeros[xt.shape[1]])

  def cheb_T(xt):
    P1 = spmm(xt)
    T1 = _make_cheb_first(xt.shape[1])(P1)
    P2 = spmm(T1)
    T2 = _make_cheb_next(xt.shape[1])(P2, xt)
    P3 = spmm(T2)
    return xt, T1, T2, P3

  def conv(xt, rows, C, W, bias, shortcut="none", S=None, Ws=None, bs=None,
           relu=False, stats=True):
    T0, T1, T2, P3 = cheb_T(xt)
    F = W.shape[2]
    args = [T0.reshape(rows, C), T1.reshape(rows, C), T2.reshape(rows, C),
            P3.reshape(2, rows, C), W, bias.reshape(1, F)]
    if shortcut == "proj":
      args += [S, Ws, bs.reshape(1, F)]
    elif shortcut == "id":
      args += [S]
    res = _make_conv_out(rows, C, F, shortcut, 0 if S is None else S.shape[1],
                         relu, stats)(*args)
    return res if stats else res[0]

  def bn_relu(h, ss, sq, g, b):
    C = h.shape[1]
    return _make_bn_relu(C)(h, ss, sq, g.reshape(1, C), b.reshape(1, C))

  xt16 = jnp.pad(jnp.transpose(x, (2, 0, 1)), ((0, 0), (0, 0), (0, 1))).reshape(N, 16)
  W0p = jnp.pad(p['conv0_W'], ((0, 0), (0, 1), (0, 0)))

  out0, s0, q0 = conv(xt16, NB, 4, W0p, p['conv0_b'], relu=True)

  a = bn_relu(out0, s0, q0, p['b1_bn1_g'], p['b1_bn1_b'])
  h1, hs, hq = conv(a.reshape(N, 64), NB, 16, p['b1_W1'], p['b1_b1'])
  a2 = bn_relu(h1, hs, hq, p['b1_bn2_g'], p['b1_bn2_b'])
  x1, s1, q1 = conv(a2.reshape(N, 64), NB, 16, p['b1_W2'], p['b1_b2'],
                    shortcut="id", S=out0)

  a = bn_relu(x1, s1, q1, p['b2_bn1_g'], p['b2_bn1_b'])
  h1, hs, hq = conv(a.reshape(N, 64), NB, 16, p['b2_W1'], p['b2_b1'])
  a2 = bn_relu(h1, hs, hq, p['b2_bn2_g'], p['b2_bn2_b'])
  x2, s2, q2 = conv(a2.reshape(N, 128), NB, 32, p['b2_W2'], p['b2_b2'],
                    shortcut="proj", S=a, Ws=p['b2_Ws'], bs=p['b2_bs'])

  a = bn_relu(x2, s2, q2, p['b3_bn1_g'], p['b3_bn1_b'])
  h1, hs, hq = conv(a.reshape(N, 128), NB, 32, p['b3_W1'], p['b3_b1'])
  a2 = bn_relu(h1, hs, hq, p['b3_bn2_g'], p['b3_bn2_b'])

  a2r = a2.reshape(N, B, 64)
  ar = a.reshape(N, B, 32)
  ybs = []
  for h in range(2):
    xt_h = a2r[:, 2 * h:2 * h + 2, :].reshape(N, 128)
    S_h = ar[:, 2 * h:2 * h + 2, :].reshape(2 * N, 32)
    y_h = conv(xt_h, 2 * N, 64, p['b3_W2'], p['b3_b2'],
               shortcut="proj", S=S_h, Ws=p['b3_Ws'], bs=p['b3_bs'], stats=False)
    yr = y_h.reshape(N, 2, 64)
    ybs += [yr[:, 0, :], yr[:, 1, :]]

  return _make_head()(ybs[0], ybs[1], ybs[2], ybs[3],
                      p['fc_W'], p['fc_b'].reshape(1, 10))
